# Initial kernel scaffold; baseline (speedup 1.0000x reference)
import jax
import jax.numpy as jnp
from jax.experimental import pallas as pl


def kernel(x, edge_index, edge_weight, deg_inv):
    raise NotImplementedError("write your pallas kernel here")


# speedup vs baseline: 4.0860x; 4.0860x over previous
import functools

import jax
import jax.numpy as jnp
from jax import lax
from jax.experimental import pallas as pl
from jax.experimental.pallas import tpu as pltpu
from jax.experimental.pallas import tpu_sc as plsc

N = 10000
D = 128
E = 320000
NC, NS = 2, 16
NW = NC * NS
C = 128
CHUNKS = -(-E // (NW * C))
EPT = CHUNKS * C
E_PAD = NW * EPT
N_PAD = 10240
NPT = N_PAD // NS
ZCH = 128


def _sc_body(row_hbm, col_hbm, w_hbm, x_hbm, parts_hbm,
             acc, rowv, colv, wv, rows, sem):
    c = lax.axis_index("c")
    s = lax.axis_index("s")
    wid = c * NS + s

    zero16 = jnp.zeros((16,), jnp.float32)

    def zero_row(r, carry):
        for j in range(D // 16):
            rows[r, pl.ds(j * 16, 16)] = zero16
        return carry

    lax.fori_loop(0, ZCH, zero_row, 0)

    for k in range(NPT // ZCH):
        pltpu.sync_copy(rows.at[pl.ds(0, ZCH)],
                        acc.at[pl.ds(s * NPT + k * ZCH, ZCH)])
    plsc.subcore_barrier()

    pltpu.sync_copy(row_hbm.at[wid], rowv)
    pltpu.sync_copy(col_hbm.at[wid], colv)
    pltpu.sync_copy(w_hbm.at[wid], wv)

    def chunk_body(i, carry):
        pltpu.async_copy(x_hbm.at[colv.at[i]], rows, sem).wait()

        def scale_group(g, carry2):
            w16 = wv[i, pl.ds(g * 16, 16)]
            for e in range(16):
                wb = lax.broadcast(w16[e], (16,))
                for j in range(D // 16):
                    sl = pl.ds(j * 16, 16)
                    rows[g * 16 + e, sl] = rows[g * 16 + e, sl] * wb
            return carry2

        lax.fori_loop(0, C // 16, scale_group, 0)
        pltpu.sync_copy(rows, acc.at[rowv.at[i]], add=True)
        return carry

    lax.fori_loop(0, CHUNKS, chunk_body, 0)
    plsc.subcore_barrier()

    for k in range(NPT // ZCH):
        rb = s * NPT + k * ZCH
        pltpu.sync_copy(acc.at[pl.ds(rb, ZCH)], rows.at[pl.ds(0, ZCH)])
        pltpu.sync_copy(rows.at[pl.ds(0, ZCH)], parts_hbm.at[c, pl.ds(rb, ZCH)])


def _sc_scatter(row3, col3, w3, x):
    mesh = plsc.VectorSubcoreMesh(core_axis_name="c", subcore_axis_name="s",
                                  num_cores=NC, num_subcores=NS)
    return pl.kernel(
        _sc_body,
        out_type=jax.ShapeDtypeStruct((NC, N_PAD, D), jnp.float32),
        mesh=mesh,
        scratch_types=[
            pltpu.VMEM_SHARED((N_PAD, D), jnp.float32),
            pltpu.VMEM((CHUNKS, C), jnp.int32),
            pltpu.VMEM((CHUNKS, C), jnp.int32),
            pltpu.VMEM((CHUNKS, C), jnp.float32),
            pltpu.VMEM((C, D), jnp.float32),
            pltpu.SemaphoreType.DMA,
        ],
    )(row3, col3, w3, x)


def _combine_body(p_ref, d_ref, o_ref):
    o_ref[...] = (p_ref[0] + p_ref[1]) * d_ref[...]


def _combine(parts, deg2d):
    bn = 2000
    return pl.pallas_call(
        _combine_body,
        out_shape=jax.ShapeDtypeStruct((N, D), jnp.float32),
        grid=(N // bn,),
        in_specs=[
            pl.BlockSpec((NC, bn, D), lambda i: (0, i, 0)),
            pl.BlockSpec((bn, 1), lambda i: (i, 0)),
        ],
        out_specs=pl.BlockSpec((bn, D), lambda i: (i, 0)),
    )(parts, deg2d)


def kernel(x, edge_index, edge_weight, deg_inv):
    row = edge_index[0].astype(jnp.int32)
    col = edge_index[1].astype(jnp.int32)
    w = edge_weight.astype(jnp.float32)
    pad = E_PAD - E
    row3 = jnp.concatenate([row, jnp.zeros((pad,), jnp.int32)]).reshape(NW, CHUNKS, C)
    col3 = jnp.concatenate([col, jnp.zeros((pad,), jnp.int32)]).reshape(NW, CHUNKS, C)
    w3 = jnp.concatenate([w, jnp.zeros((pad,), jnp.float32)]).reshape(NW, CHUNKS, C)
    parts = _sc_scatter(row3, col3, w3, x)
    return _combine(parts, deg_inv[:, None])
